# baseline (speedup 1.0000x reference)
import functools

import jax
import jax.numpy as jnp
from jax import lax
from jax.experimental import pallas as pl
from jax.experimental.pallas import tpu as pltpu
from jax.experimental.pallas import tpu_sc as plsc

EMBEDDING_DIM = 16

_info = plsc.get_sparse_core_info()
_NC, _NS = _info.num_cores, _info.num_subcores
_NW = _NC * _NS
_BATCH = 16384
_HIST = 50
_BPW = _BATCH // _NW
_TCW = _BPW // 128

_mesh = plsc.VectorSubcoreMesh(core_axis_name="c", subcore_axis_name="s")


@functools.partial(
    pl.kernel,
    mesh=_mesh,
    out_type=jax.ShapeDtypeStruct((_HIST, 2, 128, 8, 128), jnp.float32),
    scratch_types=[
        pltpu.VMEM((_HIST, _BPW), jnp.int32),
        pltpu.VMEM((_BPW, EMBEDDING_DIM), jnp.float32),
        pltpu.VMEM((_BPW, EMBEDDING_DIM), jnp.float32),
        pltpu.VMEM((2, _TCW, 8, 128), jnp.float32),
        pltpu.VMEM((2, _TCW, 8, 128), jnp.float32),
        pltpu.SemaphoreType.DMA,
        pltpu.SemaphoreType.DMA,
    ],
    compiler_params=pltpu.CompilerParams(
        use_tc_tiling_on_sc=False, needs_layout_passes=False
    ),
)
def _embed(xt_hbm, table_hbm, out_hbm, idx_v, rows_a, rows_b, dst_a, dst_b,
           sem_g, sem_s):
    rows_v = (rows_a, rows_b)
    dst_v = (dst_a, dst_b)
    wid = lax.axis_index("s") * _NC + lax.axis_index("c")
    b0 = wid * _BPW
    tc0 = wid * _TCW

    pltpu.sync_copy(xt_hbm.at[:, pl.ds(b0, _BPW)], idx_v)

    def start_gather(h, buf):
        return pltpu.async_copy(
            table_hbm.at[idx_v.at[h]], rows_v[buf], sem_g
        )

    def start_store(h, buf):
        return pltpu.async_copy(
            dst_v[buf], out_hbm.at[h, :, pl.ds(tc0, _TCW)], sem_s
        )

    iota = lax.iota(jnp.int32, 16)

    def transpose(buf):
        rows = rows_v[buf]
        dst = dst_v[buf]
        for tcl in range(_TCW):
            irs = [tcl * 128 + cb * 16 + iota for cb in range(8)]
            for tr in range(2):
                for r in range(8):
                    col = jnp.full((16,), 8 * tr + r, jnp.int32)
                    for cb in range(8):
                        v = plsc.load_gather(rows, [irs[cb], col])
                        dst[tr, tcl, r, pl.ds(cb * 16, 16)] = v

    def drain_gather(buf):
        pltpu.make_async_copy(
            table_hbm.at[pl.ds(0, _BPW)], rows_v[buf], sem_g
        ).wait()

    def drain_store(buf):
        pltpu.make_async_copy(
            out_hbm.at[0, :, pl.ds(tc0, _TCW)], dst_v[buf], sem_s
        ).wait()

    start_gather(0, 0)
    start_gather(1, 1)

    def pair(i, carry):
        for b in range(2):
            h = 2 * i + b
            drain_gather(b)

            @pl.when(h >= 2)
            def _():
                drain_store(b)

            transpose(b)
            start_store(h, b)

            @pl.when(h + 2 < _HIST)
            def _():
                start_gather(h + 2, b)

        return carry

    lax.fori_loop(0, _HIST // 2, pair, 0)
    drain_store(0)
    drain_store(1)


def kernel(x, table):
    out5 = _embed(x.T, table)
    return jnp.transpose(out5, (2, 4, 0, 1, 3)).reshape(_BATCH, _HIST, EMBEDDING_DIM)


# speedup vs baseline: 1.0749x; 1.0749x over previous
import functools

import jax
import jax.numpy as jnp
from jax import lax
from jax.experimental import pallas as pl
from jax.experimental.pallas import tpu as pltpu
from jax.experimental.pallas import tpu_sc as plsc

EMBEDDING_DIM = 16

_info = plsc.get_sparse_core_info()
_NC, _NS = _info.num_cores, _info.num_subcores
_NW = _NC * _NS
_BATCH = 16384
_HIST = 50
_BPW = _BATCH // _NW
_TCW = _BPW // 128

_mesh = plsc.VectorSubcoreMesh(core_axis_name="c", subcore_axis_name="s")


@functools.partial(
    pl.kernel,
    mesh=_mesh,
    out_type=jax.ShapeDtypeStruct((_HIST, 2, 128, 8, 128), jnp.float32),
    scratch_types=[
        pltpu.VMEM((_HIST, _BPW), jnp.int32),
        pltpu.VMEM((_BPW, EMBEDDING_DIM), jnp.float32),
        pltpu.VMEM((_BPW, EMBEDDING_DIM), jnp.float32),
        pltpu.VMEM((2, _TCW, 8, 128), jnp.float32),
        pltpu.VMEM((2, _TCW, 8, 128), jnp.float32),
        pltpu.SemaphoreType.DMA,
        pltpu.SemaphoreType.DMA,
    ],
    compiler_params=pltpu.CompilerParams(
        use_tc_tiling_on_sc=False, needs_layout_passes=False
    ),
)
def _embed(xt_hbm, table_hbm, out_hbm, idx_v, rows_a, rows_b, dst_a, dst_b,
           sem_g, sem_s):
    rows_v = (rows_a, rows_b)
    dst_v = (dst_a, dst_b)
    wid = lax.axis_index("s") * _NC + lax.axis_index("c")
    b0 = wid * _BPW
    tc0 = wid * _TCW

    pltpu.sync_copy(xt_hbm.at[:, pl.ds(b0, _BPW)], idx_v)

    def start_gather(h, buf):
        return pltpu.async_copy(
            table_hbm.at[idx_v.at[h]], rows_v[buf], sem_g
        )

    def start_store(h, buf):
        return pltpu.async_copy(
            dst_v[buf], out_hbm.at[h, :, pl.ds(tc0, _TCW)], sem_s
        )

    iota = lax.iota(jnp.int32, 16)

    def transpose(buf):
        rows = rows_v[buf]
        dst = dst_v[buf]

        def body(u, carry):
            tr = u >> 5
            tcl = (u >> 3) & (_TCW - 1)
            r = u & 7
            col = jnp.full((16,), 8 * tr + r, jnp.int32)
            rbase = tcl * 128
            for cb in range(8):
                ir = rbase + cb * 16 + iota
                v = plsc.load_gather(rows, [ir, col])
                dst[tr, tcl, r, pl.ds(cb * 16, 16)] = v
            return carry

        lax.fori_loop(0, 2 * _TCW * 8, body, 0)

    g = start_gather(0, 0)
    stores = [None] * _HIST
    for h in range(_HIST):
        buf = h & 1
        g.wait()
        if h + 1 < _HIST:
            g = start_gather(h + 1, 1 - buf)
        if h >= 2:
            stores[h - 2].wait()
        transpose(buf)
        stores[h] = start_store(h, buf)
    stores[_HIST - 2].wait()
    stores[_HIST - 1].wait()


def kernel(x, table):
    out5 = _embed(x.T, table)
    return jnp.transpose(out5, (2, 4, 0, 1, 3)).reshape(_BATCH, _HIST, EMBEDDING_DIM)
